# baseline (speedup 1.0000x reference)
import jax
import jax.numpy as jnp
from jax.experimental import pallas as pl


_ROWS_PER_BLOCK = 256
_CHANNELS = 2048
_LANE = 128
_NBLK = _CHANNELS // _LANE


def _shuffle_block(off_ref, blk_ref, x_ref, o_ref):
    r = x_ref.shape[0]
    off = jnp.broadcast_to(off_ref[...], (r, _CHANNELS))
    acc = jnp.zeros((r, _CHANNELS), jnp.float32)
    for j in range(_NBLK):
        src = x_ref[:, j * _LANE:(j + 1) * _LANE]
        gathered = jnp.take_along_axis(src, off, axis=1)
        acc = jnp.where(blk_ref[...] == j, gathered, acc)
    o_ref[...] = acc


def kernel(x, forward_permutation):
    b, s, c = x.shape
    rows = b * s
    x2 = x.reshape(rows, c)
    perm2 = forward_permutation.reshape(1, c)
    off = (perm2 % _LANE).astype(jnp.int32)
    blk = (perm2 // _LANE).astype(jnp.int32)
    grid = (rows // _ROWS_PER_BLOCK,)
    out = pl.pallas_call(
        _shuffle_block,
        grid=grid,
        in_specs=[
            pl.BlockSpec((1, c), lambda i: (0, 0)),
            pl.BlockSpec((1, c), lambda i: (0, 0)),
            pl.BlockSpec((_ROWS_PER_BLOCK, c), lambda i: (i, 0)),
        ],
        out_specs=pl.BlockSpec((_ROWS_PER_BLOCK, c), lambda i: (i, 0)),
        out_shape=jax.ShapeDtypeStruct((rows, c), x.dtype),
    )(off, blk, x2)
    return out.reshape(b, s, c)


# speedup vs baseline: 2.1625x; 2.1625x over previous
import functools

import jax
import jax.numpy as jnp
from jax import lax
from jax.experimental import pallas as pl
from jax.experimental.pallas import tpu as pltpu
from jax.experimental.pallas import tpu_sc as plsc

_C = 2048
_L = 16
_NC, _NS = 2, 16
_NW = _NC * _NS
_G = 8


def _make_sc_kernel(rows):
    rows_per_tile = rows // _NW
    num_blocks = rows_per_tile // _G
    mesh = plsc.VectorSubcoreMesh(
        core_axis_name="c", subcore_axis_name="s",
        num_cores=_NC, num_subcores=_NS)

    @functools.partial(
        pl.kernel,
        out_type=jax.ShapeDtypeStruct((rows * _C,), jnp.float32),
        mesh=mesh,
        compiler_params=pltpu.CompilerParams(needs_layout_passes=False),
        scratch_types=[
            pltpu.VMEM((_C,), jnp.int32),
            pltpu.VMEM((_G * _C,), jnp.float32),
            pltpu.VMEM((_G * _C,), jnp.float32),
        ],
    )
    def run(x_hbm, perm_hbm, out_hbm, perm_v, in_v, out_v):
        wid = lax.axis_index("s") * _NC + lax.axis_index("c")
        pltpu.sync_copy(perm_hbm, perm_v)
        tile_base = wid * (rows_per_tile * _C)

        def block_body(i, carry):
            off = tile_base + i * (_G * _C)
            pltpu.sync_copy(x_hbm.at[pl.ds(off, _G * _C)], in_v)

            def chunk_body(cc, carry2):
                idxv = perm_v[pl.ds(cc * _L, _L)]
                for g in range(_G):
                    v = plsc.load_gather(in_v, [idxv + g * _C])
                    out_v[pl.ds(g * _C + cc * _L, _L)] = v
                return carry2

            lax.fori_loop(0, _C // _L, chunk_body, 0, unroll=False)
            pltpu.sync_copy(out_v, out_hbm.at[pl.ds(off, _G * _C)])
            return carry

        lax.fori_loop(0, num_blocks, block_body, 0, unroll=False)

    return run


def kernel(x, forward_permutation):
    b, s, c = x.shape
    rows = b * s
    x_flat = x.reshape(rows * c)
    run = _make_sc_kernel(rows)
    out = run(x_flat, forward_permutation.astype(jnp.int32))
    return out.reshape(b, s, c)


# speedup vs baseline: 4.3505x; 2.0118x over previous
import functools

import jax
import jax.numpy as jnp
from jax import lax
from jax.experimental import pallas as pl
from jax.experimental.pallas import tpu as pltpu
from jax.experimental.pallas import tpu_sc as plsc

_C = 2048
_L = 16
_NC, _NS = 2, 16
_NW = _NC * _NS
_G = 8
_W = _G * _C


def _make_sc_kernel(rows):
    rows_per_tile = rows // _NW
    num_blocks = rows_per_tile // _G
    mesh = plsc.VectorSubcoreMesh(
        core_axis_name="c", subcore_axis_name="s",
        num_cores=_NC, num_subcores=_NS)

    @functools.partial(
        pl.kernel,
        out_type=jax.ShapeDtypeStruct((rows * _C,), jnp.float32),
        mesh=mesh,
        compiler_params=pltpu.CompilerParams(needs_layout_passes=False),
        scratch_types=[
            pltpu.VMEM((_C,), jnp.int32),
            pltpu.VMEM((_W,), jnp.float32),
            pltpu.VMEM((_W,), jnp.float32),
            pltpu.VMEM((_W,), jnp.float32),
            pltpu.VMEM((_W,), jnp.float32),
            pltpu.SemaphoreType.DMA,
            pltpu.SemaphoreType.DMA,
            pltpu.SemaphoreType.DMA,
            pltpu.SemaphoreType.DMA,
        ],
    )
    def run(x_hbm, perm_hbm, out_hbm, perm_v,
            in0, in1, out0, out1, isem0, isem1, osem0, osem1):
        wid = lax.axis_index("s") * _NC + lax.axis_index("c")
        pltpu.sync_copy(perm_hbm, perm_v)
        tile_base = wid * (rows_per_tile * _C)
        ins, outs = (in0, in1), (out0, out1)
        isems, osems = (isem0, isem1), (osem0, osem1)

        def off(blk):
            return tile_base + blk * _W

        pltpu.async_copy(x_hbm.at[pl.ds(off(0), _W)], ins[0], isems[0])

        def pair_body(i2, carry):
            for b in range(2):
                blk = i2 * 2 + b
                pltpu.make_async_copy(
                    x_hbm.at[pl.ds(off(blk), _W)], ins[b], isems[b]).wait()

                @pl.when(blk + 1 < num_blocks)
                def _prefetch():
                    pltpu.async_copy(
                        x_hbm.at[pl.ds(off(blk + 1), _W)],
                        ins[1 - b], isems[1 - b])

                @pl.when(blk >= 2)
                def _drain_prev():
                    pltpu.make_async_copy(
                        outs[b], out_hbm.at[pl.ds(off(blk - 2), _W)],
                        osems[b]).wait()

                @plsc.parallel_loop(0, _C // _L)
                def _chunk(cc):
                    idxv = perm_v[pl.ds(cc * _L, _L)]
                    for g in range(_G):
                        v = plsc.load_gather(ins[b], [idxv + g * _C])
                        outs[b][pl.ds(g * _C + cc * _L, _L)] = v

                pltpu.async_copy(
                    outs[b], out_hbm.at[pl.ds(off(blk), _W)], osems[b])
            return carry

        lax.fori_loop(0, num_blocks // 2, pair_body, 0, unroll=False)
        pltpu.make_async_copy(
            outs[0], out_hbm.at[pl.ds(off(num_blocks - 2), _W)],
            osems[0]).wait()
        pltpu.make_async_copy(
            outs[1], out_hbm.at[pl.ds(off(num_blocks - 1), _W)],
            osems[1]).wait()

    return run


def kernel(x, forward_permutation):
    b, s, c = x.shape
    rows = b * s
    x_flat = x.reshape(rows * c)
    run = _make_sc_kernel(rows)
    out = run(x_flat, forward_permutation.astype(jnp.int32))
    return out.reshape(b, s, c)


# speedup vs baseline: 12.7981x; 2.9417x over previous
import functools

import jax
import jax.numpy as jnp
from jax import lax
from jax.experimental import pallas as pl
from jax.experimental.pallas import tpu as pltpu
from jax.experimental.pallas import tpu_sc as plsc

_C = 2048
_L = 16
_NC, _NS = 2, 16
_NW = _NC * _NS
_G = 8


def _make_sc_kernel(rows):
    rows_per_tile = rows // _NW
    num_blocks = rows_per_tile // _G
    mesh = plsc.VectorSubcoreMesh(
        core_axis_name="c", subcore_axis_name="s",
        num_cores=_NC, num_subcores=_NS)

    @functools.partial(
        pl.kernel,
        out_type=jax.ShapeDtypeStruct((rows, _C), jnp.float32),
        mesh=mesh,
        compiler_params=pltpu.CompilerParams(needs_layout_passes=False),
        scratch_types=[
            pltpu.VMEM((_C,), jnp.int32),
            pltpu.VMEM((_G, _C), jnp.float32),
            pltpu.VMEM((_G, _C), jnp.float32),
            pltpu.VMEM((_G, _C), jnp.float32),
            pltpu.VMEM((_G, _C), jnp.float32),
            pltpu.SemaphoreType.DMA,
            pltpu.SemaphoreType.DMA,
            pltpu.SemaphoreType.DMA,
            pltpu.SemaphoreType.DMA,
        ],
    )
    def run(x_hbm, perm_hbm, out_hbm, perm_v,
            in0, in1, out0, out1, isem0, isem1, osem0, osem1):
        wid = lax.axis_index("s") * _NC + lax.axis_index("c")
        pltpu.sync_copy(perm_hbm, perm_v)
        tile_base = wid * rows_per_tile
        ins, outs = (in0, in1), (out0, out1)
        isems, osems = (isem0, isem1), (osem0, osem1)

        def row0(blk):
            return tile_base + blk * _G

        pltpu.async_copy(x_hbm.at[pl.ds(row0(0), _G), :], ins[0], isems[0])

        def pair_body(i2, carry):
            for b in range(2):
                blk = i2 * 2 + b
                pltpu.make_async_copy(
                    x_hbm.at[pl.ds(row0(blk), _G), :], ins[b],
                    isems[b]).wait()

                @pl.when(blk + 1 < num_blocks)
                def _prefetch():
                    pltpu.async_copy(
                        x_hbm.at[pl.ds(row0(blk + 1), _G), :],
                        ins[1 - b], isems[1 - b])

                @pl.when(blk >= 2)
                def _drain_prev():
                    pltpu.make_async_copy(
                        outs[b], out_hbm.at[pl.ds(row0(blk - 2), _G), :],
                        osems[b]).wait()

                @plsc.parallel_loop(0, _C // _L)
                def _chunk(cc):
                    idxv = perm_v[pl.ds(cc * _L, _L)]
                    for g in range(_G):
                        v = plsc.load_gather(
                            ins[b], [jnp.full((_L,), g, jnp.int32), idxv])
                        outs[b][g, pl.ds(cc * _L, _L)] = v

                pltpu.async_copy(
                    outs[b], out_hbm.at[pl.ds(row0(blk), _G), :], osems[b])
            return carry

        lax.fori_loop(0, num_blocks // 2, pair_body, 0, unroll=False)
        pltpu.make_async_copy(
            outs[0], out_hbm.at[pl.ds(row0(num_blocks - 2), _G), :],
            osems[0]).wait()
        pltpu.make_async_copy(
            outs[1], out_hbm.at[pl.ds(row0(num_blocks - 1), _G), :],
            osems[1]).wait()

    return run


def kernel(x, forward_permutation):
    b, s, c = x.shape
    rows = b * s
    x2 = x.reshape(rows, c)
    run = _make_sc_kernel(rows)
    out = run(x2, forward_permutation.astype(jnp.int32))
    return out.reshape(b, s, c)
